# Initial kernel scaffold; baseline (speedup 1.0000x reference)
import jax
import jax.numpy as jnp
from jax.experimental import pallas as pl


def kernel(x, molecule_idx):
    raise NotImplementedError("write your pallas kernel here")


# speedup vs baseline: 4.6536x; 4.6536x over previous
import functools

import jax
import jax.numpy as jnp
from jax import lax
from jax.experimental import pallas as pl
from jax.experimental.pallas import tpu as pltpu
from jax.experimental.pallas import tpu_sc as plsc

N_ROWS = 320000
N_FEAT = 128
N_SEG = 10000

NC = 2
NS = 16
NW = NC * NS

ROWS_PER_W = N_ROWS // NW
CHUNK = 80
N_CHUNKS = ROWS_PER_W // CHUNK

SEG_PER_W = 320
SEG_LAST_OWN = N_SEG - (NW - 1) * SEG_PER_W
SEG_BUF = SEG_PER_W

_mesh = plsc.VectorSubcoreMesh(core_axis_name="c", subcore_axis_name="s")


@functools.partial(
    pl.kernel,
    out_type=[
        jax.ShapeDtypeStruct((NC, N_SEG, N_FEAT), jnp.float32),
        jax.ShapeDtypeStruct((N_SEG,), jnp.float32),
        jax.ShapeDtypeStruct((N_SEG,), jnp.float32),
    ],
    mesh=_mesh,
    scratch_types=[
        pltpu.VMEM((CHUNK, N_FEAT), jnp.float32),
        pltpu.VMEM((CHUNK,), jnp.int32),
        pltpu.VMEM((CHUNK, N_FEAT), jnp.float32),
        pltpu.VMEM((640,), jnp.float32),
        pltpu.VMEM((CHUNK,), jnp.float32),
        pltpu.VMEM_SHARED((N_SEG, N_FEAT), jnp.float32),
        pltpu.VMEM_SHARED((N_SEG,), jnp.float32),
    ],
)
def _phase_a(x_hbm, ids_hbm, psums_hbm, pc0_hbm, pc1_hbm,
             rows_v, idx_v, zeros_v, zeros1d, ones_v, sums_sh, counts_sh):
    cid = lax.axis_index("c")
    sid = lax.axis_index("s")
    wid = sid * NC + cid

    zvec = jnp.zeros((16,), jnp.float32)

    def _fill_zrow(r, _):
        for j in range(N_FEAT // 16):
            zeros_v[r, pl.ds(j * 16, 16)] = zvec
        return _
    lax.fori_loop(0, CHUNK, _fill_zrow, None)

    def _fill_z1d(i, _):
        zeros1d[pl.ds(i * 16, 16)] = zvec
        return _
    lax.fori_loop(0, 640 // 16, _fill_z1d, None)

    for i in range(CHUNK // 16):
        ones_v[pl.ds(i * 16, 16)] = jnp.ones((16,), jnp.float32)

    def _zero_chunk(j, _):
        c = sid + j * NS

        @pl.when(c < N_CHUNKS)
        def _():
            pltpu.sync_copy(zeros_v, sums_sh.at[pl.ds(c * CHUNK, CHUNK)])
        return _
    lax.fori_loop(0, (N_CHUNKS + NS - 1) // NS, _zero_chunk, None)

    CSTRIPE = 632

    @pl.when(sid < NS - 1)
    def _():
        pltpu.sync_copy(zeros1d.at[pl.ds(0, CSTRIPE)],
                        counts_sh.at[pl.ds(sid * CSTRIPE, CSTRIPE)])

    @pl.when(sid == NS - 1)
    def _():
        rem = N_SEG - (NS - 1) * CSTRIPE
        pltpu.sync_copy(zeros1d.at[pl.ds(0, rem)],
                        counts_sh.at[pl.ds((NS - 1) * CSTRIPE, rem)])

    plsc.subcore_barrier()

    def _chunk(i, _):
        base = wid * ROWS_PER_W + i * CHUNK
        pltpu.sync_copy(x_hbm.at[pl.ds(base, CHUNK)], rows_v)
        pltpu.sync_copy(ids_hbm.at[pl.ds(base, CHUNK)], idx_v)
        pltpu.sync_copy(rows_v, sums_sh.at[idx_v], add=True)
        pltpu.sync_copy(ones_v, counts_sh.at[idx_v], add=True)
        return _
    lax.fori_loop(0, N_CHUNKS, _chunk, None)

    plsc.subcore_barrier()

    SSTRIPE = 624

    @pl.when(sid < NS - 1)
    def _():
        pltpu.sync_copy(sums_sh.at[pl.ds(sid * SSTRIPE, SSTRIPE)],
                        psums_hbm.at[cid, pl.ds(sid * SSTRIPE, SSTRIPE)])

    @pl.when(sid == NS - 1)
    def _():
        srem = N_SEG - (NS - 1) * SSTRIPE
        pltpu.sync_copy(sums_sh.at[pl.ds((NS - 1) * SSTRIPE, srem)],
                        psums_hbm.at[cid, pl.ds((NS - 1) * SSTRIPE, srem)])

    @pl.when(jnp.logical_and(sid == 0, cid == 0))
    def _():
        pltpu.sync_copy(counts_sh, pc0_hbm)

    @pl.when(jnp.logical_and(sid == 0, cid == 1))
    def _():
        pltpu.sync_copy(counts_sh, pc1_hbm)


@functools.partial(
    pl.kernel,
    out_type=jax.ShapeDtypeStruct((N_SEG, N_FEAT), jnp.float32),
    mesh=_mesh,
    scratch_types=[
        pltpu.VMEM((SEG_BUF, N_FEAT), jnp.float32),
        pltpu.VMEM((SEG_BUF, N_FEAT), jnp.float32),
        pltpu.VMEM((SEG_BUF,), jnp.float32),
        pltpu.VMEM((SEG_BUF,), jnp.float32),
    ],
)
def _phase_b(psums_hbm, pc0_hbm, pc1_hbm, out_hbm, s0, s1, c0, c1):
    cid = lax.axis_index("c")
    sid = lax.axis_index("s")
    wid = sid * NC + cid
    own = wid * SEG_PER_W
    is_last = wid == NW - 1
    base = jnp.where(is_last, N_SEG - SEG_BUF, own)
    off = jnp.where(is_last, SEG_BUF - SEG_LAST_OWN, 0)

    pltpu.sync_copy(psums_hbm.at[0, pl.ds(base, SEG_BUF)], s0)
    pltpu.sync_copy(psums_hbm.at[1, pl.ds(base, SEG_BUF)], s1)
    pltpu.sync_copy(pc0_hbm.at[pl.ds(base, SEG_BUF)], c0)
    pltpu.sync_copy(pc1_hbm.at[pl.ds(base, SEG_BUF)], c1)

    def _group(g, _):
        cnt = c0[pl.ds(g * 16, 16)] + c1[pl.ds(g * 16, 16)]
        r = 1.0 / jnp.maximum(cnt, 1.0)
        for k in range(16):
            rk = r[k]
            row = g * 16 + k
            for j in range(N_FEAT // 16):
                a = s0[row, pl.ds(j * 16, 16)]
                b = s1[row, pl.ds(j * 16, 16)]
                s0[row, pl.ds(j * 16, 16)] = (a + b) * rk
        return _
    lax.fori_loop(0, SEG_BUF // 16, _group, None)

    @pl.when(jnp.logical_not(is_last))
    def _():
        pltpu.sync_copy(s0.at[pl.ds(0, SEG_PER_W)],
                        out_hbm.at[pl.ds(own, SEG_PER_W)])

    @pl.when(is_last)
    def _():
        pltpu.sync_copy(s0.at[pl.ds(SEG_BUF - SEG_LAST_OWN, SEG_LAST_OWN)],
                        out_hbm.at[pl.ds(own, SEG_LAST_OWN)])


def kernel(x, molecule_idx):
    ids = molecule_idx.astype(jnp.int32)
    psums, pc0, pc1 = _phase_a(x, ids)
    return _phase_b(psums, pc0, pc1)


# speedup vs baseline: 9.4383x; 2.0282x over previous
import functools

import jax
import jax.numpy as jnp
from jax import lax
from jax.experimental import pallas as pl
from jax.experimental.pallas import tpu as pltpu
from jax.experimental.pallas import tpu_sc as plsc

N_ROWS = 320000
N_FEAT = 128
N_SEG = 10000

NC = 2
NS = 16
NW = NC * NS

ROWS_PER_W = N_ROWS // NW
CHUNK = 80
N_CHUNKS = ROWS_PER_W // CHUNK

SEG_PER_W = 320
SEG_LAST_OWN = N_SEG - (NW - 1) * SEG_PER_W
SEG_BUF = SEG_PER_W

_mesh = plsc.VectorSubcoreMesh(core_axis_name="c", subcore_axis_name="s")


@functools.partial(
    pl.kernel,
    out_type=[
        jax.ShapeDtypeStruct((NC, N_SEG, N_FEAT), jnp.float32),
        jax.ShapeDtypeStruct((N_SEG,), jnp.float32),
        jax.ShapeDtypeStruct((N_SEG,), jnp.float32),
    ],
    mesh=_mesh,
    scratch_types=[
        [pltpu.VMEM((CHUNK, N_FEAT), jnp.float32) for _ in range(3)],
        [pltpu.VMEM((CHUNK,), jnp.int32) for _ in range(3)],
        [pltpu.SemaphoreType.DMA for _ in range(3)],
        pltpu.VMEM((CHUNK, N_FEAT), jnp.float32),
        pltpu.VMEM((640,), jnp.float32),
        pltpu.VMEM((CHUNK,), jnp.float32),
        pltpu.VMEM_SHARED((N_SEG, N_FEAT), jnp.float32),
        pltpu.VMEM_SHARED((N_SEG,), jnp.float32),
    ],
)
def _phase_a(x_hbm, ids_hbm, psums_hbm, pc0_hbm, pc1_hbm,
             rows, idx, lsem, zeros_v, zeros1d, ones_v, sums_sh, counts_sh):
    cid = lax.axis_index("c")
    sid = lax.axis_index("s")
    wid = sid * NC + cid

    def _start_load(chunk, b):
        base = wid * ROWS_PER_W + chunk * CHUNK
        pltpu.async_copy(x_hbm.at[pl.ds(base, CHUNK)], rows[b], lsem[b])
        pltpu.async_copy(ids_hbm.at[pl.ds(base, CHUNK)], idx[b], lsem[b])

    def _wait_load(chunk, b):
        base = wid * ROWS_PER_W + chunk * CHUNK
        pltpu.make_async_copy(x_hbm.at[pl.ds(base, CHUNK)], rows[b], lsem[b]).wait()
        pltpu.make_async_copy(ids_hbm.at[pl.ds(base, CHUNK)], idx[b], lsem[b]).wait()

    _start_load(0, 0)
    _start_load(1, 1)

    zvec = jnp.zeros((16,), jnp.float32)

    def _fill_zrow(r, _):
        for j in range(N_FEAT // 16):
            zeros_v[r, pl.ds(j * 16, 16)] = zvec
        return _
    lax.fori_loop(0, CHUNK, _fill_zrow, None)

    def _fill_z1d(i, _):
        zeros1d[pl.ds(i * 16, 16)] = zvec
        return _
    lax.fori_loop(0, 640 // 16, _fill_z1d, None)

    for i in range(CHUNK // 16):
        ones_v[pl.ds(i * 16, 16)] = jnp.ones((16,), jnp.float32)

    def _zero_chunk(j, _):
        c = sid + j * NS

        @pl.when(c < N_CHUNKS)
        def _():
            pltpu.sync_copy(zeros_v, sums_sh.at[pl.ds(c * CHUNK, CHUNK)])
        return _
    lax.fori_loop(0, (N_CHUNKS + NS - 1) // NS, _zero_chunk, None)

    CSTRIPE = 632

    @pl.when(sid < NS - 1)
    def _():
        pltpu.sync_copy(zeros1d.at[pl.ds(0, CSTRIPE)],
                        counts_sh.at[pl.ds(sid * CSTRIPE, CSTRIPE)])

    @pl.when(sid == NS - 1)
    def _():
        rem = N_SEG - (NS - 1) * CSTRIPE
        pltpu.sync_copy(zeros1d.at[pl.ds(0, rem)],
                        counts_sh.at[pl.ds((NS - 1) * CSTRIPE, rem)])

    plsc.subcore_barrier()

    def _chunk(i, _):
        for b in range(3):
            @pl.when(i % 3 == b)
            def _():
                _wait_load(i, b)

                @pl.when(i + 2 < N_CHUNKS)
                def _():
                    _start_load(i + 2, (b + 2) % 3)

                pltpu.sync_copy(rows[b], sums_sh.at[idx[b]], add=True)
                pltpu.sync_copy(ones_v, counts_sh.at[idx[b]], add=True)
        return _
    lax.fori_loop(0, N_CHUNKS, _chunk, None)

    plsc.subcore_barrier()

    SSTRIPE = 624

    @pl.when(sid < NS - 1)
    def _():
        pltpu.sync_copy(sums_sh.at[pl.ds(sid * SSTRIPE, SSTRIPE)],
                        psums_hbm.at[cid, pl.ds(sid * SSTRIPE, SSTRIPE)])

    @pl.when(sid == NS - 1)
    def _():
        srem = N_SEG - (NS - 1) * SSTRIPE
        pltpu.sync_copy(sums_sh.at[pl.ds((NS - 1) * SSTRIPE, srem)],
                        psums_hbm.at[cid, pl.ds((NS - 1) * SSTRIPE, srem)])

    @pl.when(jnp.logical_and(sid == 0, cid == 0))
    def _():
        pltpu.sync_copy(counts_sh, pc0_hbm)

    @pl.when(jnp.logical_and(sid == 0, cid == 1))
    def _():
        pltpu.sync_copy(counts_sh, pc1_hbm)


@functools.partial(
    pl.kernel,
    out_type=jax.ShapeDtypeStruct((N_SEG, N_FEAT), jnp.float32),
    mesh=_mesh,
    scratch_types=[
        pltpu.VMEM((SEG_BUF, N_FEAT), jnp.float32),
        pltpu.VMEM((SEG_BUF, N_FEAT), jnp.float32),
        pltpu.VMEM((SEG_BUF,), jnp.float32),
        pltpu.VMEM((SEG_BUF,), jnp.float32),
        pltpu.SemaphoreType.DMA,
    ],
)
def _phase_b(psums_hbm, pc0_hbm, pc1_hbm, out_hbm, s0, s1, c0, c1, bsem):
    cid = lax.axis_index("c")
    sid = lax.axis_index("s")
    wid = sid * NC + cid
    own = wid * SEG_PER_W
    is_last = wid == NW - 1
    base = jnp.where(is_last, N_SEG - SEG_BUF, own)
    off = jnp.where(is_last, SEG_BUF - SEG_LAST_OWN, 0)

    cp0 = pltpu.async_copy(psums_hbm.at[0, pl.ds(base, SEG_BUF)], s0, bsem)
    cp1 = pltpu.async_copy(psums_hbm.at[1, pl.ds(base, SEG_BUF)], s1, bsem)
    cp2 = pltpu.async_copy(pc0_hbm.at[pl.ds(base, SEG_BUF)], c0, bsem)
    cp3 = pltpu.async_copy(pc1_hbm.at[pl.ds(base, SEG_BUF)], c1, bsem)
    cp0.wait(); cp1.wait(); cp2.wait(); cp3.wait()

    def _group(g, _):
        cnt = c0[pl.ds(g * 16, 16)] + c1[pl.ds(g * 16, 16)]
        r = 1.0 / jnp.maximum(cnt, 1.0)
        for k in range(16):
            rk = r[k]
            row = g * 16 + k
            for j in range(N_FEAT // 16):
                a = s0[row, pl.ds(j * 16, 16)]
                b = s1[row, pl.ds(j * 16, 16)]
                s0[row, pl.ds(j * 16, 16)] = (a + b) * rk
        return _
    lax.fori_loop(0, SEG_BUF // 16, _group, None)

    @pl.when(jnp.logical_not(is_last))
    def _():
        pltpu.sync_copy(s0.at[pl.ds(0, SEG_PER_W)],
                        out_hbm.at[pl.ds(own, SEG_PER_W)])

    @pl.when(is_last)
    def _():
        pltpu.sync_copy(s0.at[pl.ds(SEG_BUF - SEG_LAST_OWN, SEG_LAST_OWN)],
                        out_hbm.at[pl.ds(own, SEG_LAST_OWN)])


def kernel(x, molecule_idx):
    ids = molecule_idx.astype(jnp.int32)
    psums, pc0, pc1 = _phase_a(x, ids)
    return _phase_b(psums, pc0, pc1)
